# baseline (speedup 1.0000x reference)
import jax
import jax.numpy as jnp
from jax import lax
from jax.experimental import pallas as pl
from jax.experimental.pallas import tpu as pltpu
from jax.experimental.pallas import tpu_sc as plsc

N = 10000
NPAD = 10240
E_RAW = 320000
B = 128
EPAD = 327680
ROWS = EPAD // B
NC, NS = 2, 16
ROWS_PER_SUB = ROWS // NS
KCH0, CHR0 = 8, 20
ROWS_PER_WORKER = ROWS // (NC * NS)
KCH1, CHR1 = 4, 20
ACC_PER_SUB = NPAD // NS
MBLK = 1000


def _sc_mesh():
    return plsc.VectorSubcoreMesh(core_axis_name="c", subcore_axis_name="s",
                                  num_cores=NC, num_subcores=NS)


def _deg_body(dst_hbm, ones_hbm, zeros_hbm, deg_hbm, acc, dst_v, ones_v, sem):
    c = lax.axis_index("c")
    s = lax.axis_index("s")
    w = s * NC + c
    pltpu.sync_copy(zeros_hbm, acc.at[pl.ds(s * ACC_PER_SUB, ACC_PER_SUB)])
    pltpu.sync_copy(ones_hbm, ones_v)
    pltpu.sync_copy(dst_hbm.at[w], dst_v)
    plsc.subcore_barrier()

    def issue(j, _):
        pltpu.async_copy(ones_v, acc.at[dst_v.at[j]], sem, add=True)
        return ()

    def drain(j, _):
        pltpu.make_async_copy(ones_v, acc.at[dst_v.at[j]], sem).wait()
        return ()

    lax.fori_loop(0, ROWS_PER_WORKER, issue, ())
    lax.fori_loop(0, ROWS_PER_WORKER, drain, ())
    plsc.subcore_barrier()
    pltpu.sync_copy(acc.at[pl.ds(s * ACC_PER_SUB, ACC_PER_SUB)],
                    deg_hbm.at[c, pl.ds(s * ACC_PER_SUB, ACC_PER_SUB)])


def _make_deg_kernel():
    return pl.kernel(
        _deg_body,
        out_type=jax.ShapeDtypeStruct((NC, NPAD, 128), jnp.float32),
        mesh=_sc_mesh(),
        scratch_types=[
            pltpu.VMEM_SHARED((NPAD, 128), jnp.float32),
            pltpu.VMEM((ROWS_PER_WORKER, B), jnp.int32),
            pltpu.VMEM((B, 128), jnp.float32),
            pltpu.SemaphoreType.DMA,
        ],
    )


def _pipelined_chunk(table_hbm, acc, src_v, dst_v, bufs, gsems, ssems, ch_r):
    def g_issue(j):
        pltpu.async_copy(table_hbm.at[src_v.at[j]], bufs[j % 2], gsems[j % 2])

    def g_wait(j):
        pltpu.make_async_copy(table_hbm.at[src_v.at[j]], bufs[j % 2],
                              gsems[j % 2]).wait()

    def s_issue(j):
        pltpu.async_copy(bufs[j % 2], acc.at[dst_v.at[j]], ssems[j % 2],
                         add=True)

    def s_wait(j):
        pltpu.make_async_copy(bufs[j % 2], acc.at[dst_v.at[j]],
                              ssems[j % 2]).wait()

    g_issue(0)
    g_issue(1)
    for j in range(ch_r):
        g_wait(j)
        s_issue(j)
        if j + 2 < ch_r:
            s_wait(j)
            g_issue(j + 2)
    s_wait(ch_r - 2)
    s_wait(ch_r - 1)


def _agg_body(kch, ch_r, col_split, table_hbm, src_hbm, dst_hbm, zeros_hbm,
              out_hbm, acc, src_v, dst_v, r0, r1, gs0, gs1, ss0, ss1):
    c = lax.axis_index("c")
    s = lax.axis_index("s")
    w = s * NC + c
    pltpu.sync_copy(zeros_hbm, acc.at[pl.ds(s * ACC_PER_SUB, ACC_PER_SUB)])
    plsc.subcore_barrier()

    def chunk(k, _):
        if col_split:
            pltpu.sync_copy(src_hbm.at[c, s, k], src_v)
            pltpu.sync_copy(dst_hbm.at[s, k], dst_v)
        else:
            pltpu.sync_copy(src_hbm.at[w, k], src_v)
            pltpu.sync_copy(dst_hbm.at[w, k], dst_v)
        _pipelined_chunk(table_hbm, acc, src_v, dst_v, (r0, r1),
                         (gs0, gs1), (ss0, ss1), ch_r)
        return ()

    lax.fori_loop(0, kch, chunk, ())
    plsc.subcore_barrier()
    pltpu.sync_copy(acc.at[pl.ds(s * ACC_PER_SUB, ACC_PER_SUB)],
                    out_hbm.at[c, pl.ds(s * ACC_PER_SUB, ACC_PER_SUB)])


def _make_agg_kernel(kch, ch_r, col_split):
    import functools
    return pl.kernel(
        functools.partial(_agg_body, kch, ch_r, col_split),
        out_type=jax.ShapeDtypeStruct((NC, NPAD, 128), jnp.float32),
        mesh=_sc_mesh(),
        scratch_types=[
            pltpu.VMEM_SHARED((NPAD, 128), jnp.float32),
            pltpu.VMEM((ch_r, B), jnp.int32),
            pltpu.VMEM((ch_r, B), jnp.int32),
            pltpu.VMEM((B, 128), jnp.float32),
            pltpu.VMEM((B, 128), jnp.float32),
            pltpu.SemaphoreType.DMA,
            pltpu.SemaphoreType.DMA,
            pltpu.SemaphoreType.DMA,
            pltpu.SemaphoreType.DMA,
        ],
    )


def _dinv_block(deg_ref):
    deg = deg_ref[0, :, 0:1] + deg_ref[1, :, 0:1] + 1.0
    return lax.rsqrt(deg)


def _tc0_body(x_ref, w_ref, deg_ref, out_ref):
    dinv = _dinv_block(deg_ref)
    h = jnp.dot(x_ref[...], w_ref[...], preferred_element_type=jnp.float32)
    hs = h * dinv
    out_ref[0] = hs[:, :128]
    out_ref[1] = hs[:, 128:]


def _tc1_body(a_ref, t_ref, deg_ref, b0_ref, w_ref, out_ref):
    dinv = _dinv_block(deg_ref)
    agg = jnp.concatenate([a_ref[0] + t_ref[0], a_ref[1] + t_ref[1]], axis=1)
    y = jnp.maximum(agg * dinv + b0_ref[...], 0.0)
    h = jnp.dot(y, w_ref[...], preferred_element_type=jnp.float32)
    out_ref[...] = h * dinv


def _tc2_body(a_ref, t_ref, deg_ref, b1_ref, out_ref):
    dinv = _dinv_block(deg_ref)
    z = (a_ref[0] + a_ref[1] + t_ref[...]) * dinv + b1_ref[...]
    m = jnp.max(z, axis=1, keepdims=True)
    ez = jnp.exp(z - m)
    lse = jnp.log(jnp.sum(ez, axis=1, keepdims=True))
    out_ref[...] = z - m - lse


def _row_spec(shape):
    if len(shape) == 3:
        return pl.BlockSpec((shape[0], MBLK, shape[2]), lambda i: (0, i, 0))
    return pl.BlockSpec((MBLK, shape[1]), lambda i: (i, 0))


def _full_spec(shape):
    return pl.BlockSpec(shape, lambda i: tuple(0 for _ in shape))


def _tc_call(body, in_arrays, full_mask, out_shape):
    in_specs = [
        _full_spec(a.shape) if f else _row_spec(a.shape)
        for a, f in zip(in_arrays, full_mask)
    ]
    return pl.pallas_call(
        body,
        grid=(N // MBLK,),
        in_specs=in_specs,
        out_specs=_row_spec(out_shape),
        out_shape=jax.ShapeDtypeStruct(out_shape, jnp.float32),
    )(*in_arrays)


def kernel(x, edge_index, W0, b0, W1, b1):
    src = edge_index[0].astype(jnp.int32)
    dst = edge_index[1].astype(jnp.int32)
    npad = EPAD - E_RAW
    src = jnp.concatenate([src, jnp.zeros((npad,), jnp.int32)])
    dst = jnp.concatenate([dst, jnp.full((npad,), N, jnp.int32)])
    dst_w = dst.reshape(NC * NS, ROWS_PER_WORKER, B)
    dst_s = dst.reshape(NS, KCH0, CHR0, B)
    src2 = jnp.stack([src, src + N]).reshape(NC, NS, KCH0, CHR0, B)
    src_e = src.reshape(NC * NS, KCH1, CHR1, B)
    dst_e = dst.reshape(NC * NS, KCH1, CHR1, B)

    ones128 = jnp.ones((B, 128), jnp.float32)
    zeros128 = jnp.zeros((ACC_PER_SUB, 128), jnp.float32)

    deg = _make_deg_kernel()(dst_w, ones128, zeros128)
    degn = deg[:, :N, :16]

    hs0 = _tc_call(_tc0_body, [x, W0, degn], [False, True, False],
                   (2, N, 128))
    table0 = hs0.reshape(2 * N, 128)
    agg0 = _make_agg_kernel(KCH0, CHR0, True)(table0, src2, dst_s, zeros128)

    table1 = _tc_call(_tc1_body,
                      [agg0[:, :N, :], hs0, degn, b0.reshape(1, 256), W1],
                      [False, False, False, True, True], (N, 128))
    agg1 = _make_agg_kernel(KCH1, CHR1, False)(table1, src_e, dst_e, zeros128)

    out = _tc_call(_tc2_body,
                   [agg1[:, :N, :], table1, degn, b1.reshape(1, 128)],
                   [False, False, False, True], (N, 128))
    return out


# speedup vs baseline: 2.4626x; 2.4626x over previous
import jax
import jax.numpy as jnp
from jax import lax
from jax.experimental import pallas as pl
from jax.experimental.pallas import tpu as pltpu
from jax.experimental.pallas import tpu_sc as plsc

N = 10000
NPAD = 10240
E_RAW = 320000
B = 128
EPAD = 327680
ROWS = EPAD // B
NC, NS = 2, 16
ROWS_PER_SUB = ROWS // NS
KCH0, CHR0 = 8, 20
ROWS_PER_WORKER = ROWS // (NC * NS)
KCH1, CHR1 = 4, 20
ACC_PER_SUB = NPAD // NS
MBLK = 1000


def _sc_mesh():
    return plsc.VectorSubcoreMesh(core_axis_name="c", subcore_axis_name="s",
                                  num_cores=NC, num_subcores=NS)


def _deg_body(dst_hbm, ones_hbm, zeros_hbm, deg_hbm, acc, dst_v, ones_v, sem):
    c = lax.axis_index("c")
    s = lax.axis_index("s")
    w = s * NC + c
    pltpu.sync_copy(zeros_hbm, acc.at[pl.ds(s * ACC_PER_SUB, ACC_PER_SUB)])
    pltpu.sync_copy(ones_hbm, ones_v)
    pltpu.sync_copy(dst_hbm.at[w], dst_v)
    plsc.subcore_barrier()

    def issue(j, _):
        pltpu.async_copy(ones_v, acc.at[dst_v.at[j]], sem, add=True)
        return ()

    def drain(j, _):
        pltpu.make_async_copy(ones_v, acc.at[dst_v.at[j]], sem).wait()
        return ()

    lax.fori_loop(0, ROWS_PER_WORKER, issue, ())
    lax.fori_loop(0, ROWS_PER_WORKER, drain, ())
    plsc.subcore_barrier()
    pltpu.sync_copy(acc.at[pl.ds(s * ACC_PER_SUB, ACC_PER_SUB)],
                    deg_hbm.at[c, pl.ds(s * ACC_PER_SUB, ACC_PER_SUB)])


def _make_deg_kernel():
    return pl.kernel(
        _deg_body,
        out_type=jax.ShapeDtypeStruct((NC, NPAD, 128), jnp.float32),
        mesh=_sc_mesh(),
        scratch_types=[
            pltpu.VMEM_SHARED((NPAD, 128), jnp.float32),
            pltpu.VMEM((ROWS_PER_WORKER, B), jnp.int32),
            pltpu.VMEM((B, 128), jnp.float32),
            pltpu.SemaphoreType.DMA,
        ],
    )


def _pipelined_chunk(table_hbm, acc, src_v, dst_v, bufs, gsems, ssems, ch_r):
    def g_issue(j):
        pltpu.async_copy(table_hbm.at[src_v.at[j]], bufs[j % 2], gsems[j % 2])

    def g_wait(j):
        pltpu.make_async_copy(table_hbm.at[src_v.at[j]], bufs[j % 2],
                              gsems[j % 2]).wait()

    def s_issue(j):
        pltpu.async_copy(bufs[j % 2], acc.at[dst_v.at[j]], ssems[j % 2],
                         add=True)

    def s_wait(j):
        pltpu.make_async_copy(bufs[j % 2], acc.at[dst_v.at[j]],
                              ssems[j % 2]).wait()

    g_issue(0)
    g_issue(1)
    for j in range(ch_r):
        g_wait(j)
        s_issue(j)
        if j + 2 < ch_r:
            s_wait(j)
            g_issue(j + 2)
    s_wait(ch_r - 2)
    s_wait(ch_r - 1)


def _agg_body(kch, ch_r, col_split, table_hbm, src_hbm, dst_hbm, zeros_hbm,
              out_hbm, acc, src_v, dst_v, r0, r1, gs0, gs1, ss0, ss1):
    c = lax.axis_index("c")
    s = lax.axis_index("s")
    w = s * NC + c
    pltpu.sync_copy(zeros_hbm, acc.at[pl.ds(s * ACC_PER_SUB, ACC_PER_SUB)])
    plsc.subcore_barrier()

    def chunk(k, _):
        if col_split:
            pltpu.sync_copy(src_hbm.at[c, s, k], src_v)
            pltpu.sync_copy(dst_hbm.at[s, k], dst_v)
        else:
            pltpu.sync_copy(src_hbm.at[w, k], src_v)
            pltpu.sync_copy(dst_hbm.at[w, k], dst_v)
        _pipelined_chunk(table_hbm, acc, src_v, dst_v, (r0, r1),
                         (gs0, gs1), (ss0, ss1), ch_r)
        return ()

    lax.fori_loop(0, kch, chunk, ())
    plsc.subcore_barrier()
    pltpu.sync_copy(acc.at[pl.ds(s * ACC_PER_SUB, ACC_PER_SUB)],
                    out_hbm.at[c, pl.ds(s * ACC_PER_SUB, ACC_PER_SUB)])


def _make_agg_kernel(kch, ch_r, col_split):
    import functools
    return pl.kernel(
        functools.partial(_agg_body, kch, ch_r, col_split),
        out_type=jax.ShapeDtypeStruct((NC, NPAD, 128), jnp.float32),
        mesh=_sc_mesh(),
        scratch_types=[
            pltpu.VMEM_SHARED((NPAD, 128), jnp.float32),
            pltpu.VMEM((ch_r, B), jnp.int32),
            pltpu.VMEM((ch_r, B), jnp.int32),
            pltpu.VMEM((B, 128), jnp.float32),
            pltpu.VMEM((B, 128), jnp.float32),
            pltpu.SemaphoreType.DMA,
            pltpu.SemaphoreType.DMA,
            pltpu.SemaphoreType.DMA,
            pltpu.SemaphoreType.DMA,
        ],
    )


def _dinv_block(deg_ref):
    deg = deg_ref[0, :, 0:1] + deg_ref[1, :, 0:1] + 1.0
    return lax.rsqrt(deg)


def _tc0_body(x_ref, w_ref, deg_ref, out_ref):
    dinv = _dinv_block(deg_ref)
    h = jnp.dot(x_ref[...], w_ref[...], preferred_element_type=jnp.float32)
    hs = h * dinv
    out_ref[0] = hs[:, :128]
    out_ref[1] = hs[:, 128:]


def _tc1_body(a_ref, t_ref, deg_ref, b0_ref, w_ref, out_ref):
    dinv = _dinv_block(deg_ref)
    agg = jnp.concatenate([a_ref[0] + t_ref[0], a_ref[1] + t_ref[1]], axis=1)
    y = jnp.maximum(agg * dinv + b0_ref[...], 0.0)
    h = jnp.dot(y, w_ref[...], preferred_element_type=jnp.float32)
    out_ref[...] = h * dinv


def _tc2_body(a_ref, t_ref, deg_ref, b1_ref, out_ref):
    dinv = _dinv_block(deg_ref)
    z = (a_ref[0] + a_ref[1] + t_ref[...]) * dinv + b1_ref[...]
    m = jnp.max(z, axis=1, keepdims=True)
    ez = jnp.exp(z - m)
    lse = jnp.log(jnp.sum(ez, axis=1, keepdims=True))
    out_ref[...] = z - m - lse


def _row_spec(shape):
    if len(shape) == 3:
        return pl.BlockSpec((shape[0], MBLK, shape[2]), lambda i: (0, i, 0))
    return pl.BlockSpec((MBLK, shape[1]), lambda i: (i, 0))


def _full_spec(shape):
    return pl.BlockSpec(shape, lambda i: tuple(0 for _ in shape))


def _tc_call(body, in_arrays, full_mask, out_shape):
    in_specs = [
        _full_spec(a.shape) if f else _row_spec(a.shape)
        for a, f in zip(in_arrays, full_mask)
    ]
    return pl.pallas_call(
        body,
        grid=(N // MBLK,),
        in_specs=in_specs,
        out_specs=_row_spec(out_shape),
        out_shape=jax.ShapeDtypeStruct(out_shape, jnp.float32),
    )(*in_arrays)


def kernel(x, edge_index, W0, b0, W1, b1):
    src = edge_index[0].astype(jnp.int32)
    dst = edge_index[1].astype(jnp.int32)
    npad = EPAD - E_RAW
    pad_src = jnp.arange(npad, dtype=jnp.int32) % N
    pad_dst = N + (jnp.arange(npad, dtype=jnp.int32) % (NPAD - N))
    src = jnp.concatenate([src, pad_src])
    dst = jnp.concatenate([dst, pad_dst])
    dst_w = dst.reshape(NC * NS, ROWS_PER_WORKER, B)
    dst_s = dst.reshape(NS, KCH0, CHR0, B)
    src2 = jnp.stack([src, src + N]).reshape(NC, NS, KCH0, CHR0, B)
    src_e = src.reshape(NC * NS, KCH1, CHR1, B)
    dst_e = dst.reshape(NC * NS, KCH1, CHR1, B)

    ones128 = jnp.ones((B, 128), jnp.float32)
    zeros128 = jnp.zeros((ACC_PER_SUB, 128), jnp.float32)

    deg = _make_deg_kernel()(dst_w, ones128, zeros128)
    degn = deg[:, :N, :16]

    hs0 = _tc_call(_tc0_body, [x, W0, degn], [False, True, False],
                   (2, N, 128))
    table0 = hs0.reshape(2 * N, 128)
    agg0 = _make_agg_kernel(KCH0, CHR0, True)(table0, src2, dst_s, zeros128)

    table1 = _tc_call(_tc1_body,
                      [agg0[:, :N, :], hs0, degn, b0.reshape(1, 256), W1],
                      [False, False, False, True, True], (N, 128))
    agg1 = _make_agg_kernel(KCH1, CHR1, False)(table1, src_e, dst_e, zeros128)

    out = _tc_call(_tc2_body,
                   [agg1[:, :N, :], table1, degn, b1.reshape(1, 128)],
                   [False, False, False, True], (N, 128))
    return out


# speedup vs baseline: 2.4889x; 1.0107x over previous
import jax
import jax.numpy as jnp
from jax import lax
from jax.experimental import pallas as pl
from jax.experimental.pallas import tpu as pltpu
from jax.experimental.pallas import tpu_sc as plsc

N = 10000
NPAD = 10240
E_RAW = 320000
B = 112
EPAD = 322560
ROWS = EPAD // B
NC, NS = 2, 16
ROWS_PER_SUB = ROWS // NS
KCH0, CHR0 = 12, 15
ROWS_PER_WORKER = ROWS // (NC * NS)
KCH1, CHR1 = 9, 10
ACC_PER_SUB = NPAD // NS
MBLK = 1000


def _sc_mesh():
    return plsc.VectorSubcoreMesh(core_axis_name="c", subcore_axis_name="s",
                                  num_cores=NC, num_subcores=NS)


def _deg_body(dst_hbm, ones_hbm, zeros_hbm, deg_hbm, acc, dst_v, ones_v, sem):
    c = lax.axis_index("c")
    s = lax.axis_index("s")
    w = s * NC + c
    pltpu.sync_copy(zeros_hbm, acc.at[pl.ds(s * ACC_PER_SUB, ACC_PER_SUB)])
    pltpu.sync_copy(ones_hbm, ones_v)
    pltpu.sync_copy(dst_hbm.at[w], dst_v)
    plsc.subcore_barrier()

    def issue(j, _):
        pltpu.async_copy(ones_v, acc.at[dst_v.at[j]], sem, add=True)
        return ()

    def drain(j, _):
        pltpu.make_async_copy(ones_v, acc.at[dst_v.at[j]], sem).wait()
        return ()

    lax.fori_loop(0, ROWS_PER_WORKER, issue, ())
    lax.fori_loop(0, ROWS_PER_WORKER, drain, ())
    plsc.subcore_barrier()
    pltpu.sync_copy(acc.at[pl.ds(s * ACC_PER_SUB, ACC_PER_SUB)],
                    deg_hbm.at[c, pl.ds(s * ACC_PER_SUB, ACC_PER_SUB)])


def _make_deg_kernel():
    return pl.kernel(
        _deg_body,
        out_type=jax.ShapeDtypeStruct((NC, NPAD, 128), jnp.float32),
        mesh=_sc_mesh(),
        scratch_types=[
            pltpu.VMEM_SHARED((NPAD, 128), jnp.float32),
            pltpu.VMEM((ROWS_PER_WORKER, B), jnp.int32),
            pltpu.VMEM((B, 128), jnp.float32),
            pltpu.SemaphoreType.DMA,
        ],
    )


NBUF = 3


def _pipelined_chunk(table_hbm, acc, src_v, dst_v, bufs, gsems, ssems, ch_r):
    def g_issue(j):
        pltpu.async_copy(table_hbm.at[src_v.at[j]], bufs[j % NBUF],
                         gsems[j % NBUF])

    def g_wait(j):
        pltpu.make_async_copy(table_hbm.at[src_v.at[j]], bufs[j % NBUF],
                              gsems[j % NBUF]).wait()

    def s_issue(j):
        pltpu.async_copy(bufs[j % NBUF], acc.at[dst_v.at[j]], ssems[j % NBUF],
                         add=True)

    def s_wait(j):
        pltpu.make_async_copy(bufs[j % NBUF], acc.at[dst_v.at[j]],
                              ssems[j % NBUF]).wait()

    for j in range(NBUF):
        g_issue(j)
    for j in range(ch_r):
        g_wait(j)
        s_issue(j)
        if j + NBUF < ch_r:
            s_wait(j)
            g_issue(j + NBUF)
    for j in range(max(ch_r - NBUF, 0), ch_r):
        s_wait(j)


def _agg_body(kch, ch_r, col_split, table_hbm, src_hbm, dst_hbm, zeros_hbm,
              out_hbm, acc, src_v, dst_v, r0, r1, r2, gs0, gs1, gs2,
              ss0, ss1, ss2):
    c = lax.axis_index("c")
    s = lax.axis_index("s")
    w = s * NC + c
    pltpu.sync_copy(zeros_hbm, acc.at[pl.ds(s * ACC_PER_SUB, ACC_PER_SUB)])
    plsc.subcore_barrier()

    def chunk(k, _):
        if col_split:
            pltpu.sync_copy(src_hbm.at[c, s, k], src_v)
            pltpu.sync_copy(dst_hbm.at[s, k], dst_v)
        else:
            pltpu.sync_copy(src_hbm.at[w, k], src_v)
            pltpu.sync_copy(dst_hbm.at[w, k], dst_v)
        _pipelined_chunk(table_hbm, acc, src_v, dst_v, (r0, r1, r2),
                         (gs0, gs1, gs2), (ss0, ss1, ss2), ch_r)
        return ()

    lax.fori_loop(0, kch, chunk, ())
    plsc.subcore_barrier()
    pltpu.sync_copy(acc.at[pl.ds(s * ACC_PER_SUB, ACC_PER_SUB)],
                    out_hbm.at[c, pl.ds(s * ACC_PER_SUB, ACC_PER_SUB)])


def _make_agg_kernel(kch, ch_r, col_split):
    import functools
    return pl.kernel(
        functools.partial(_agg_body, kch, ch_r, col_split),
        out_type=jax.ShapeDtypeStruct((NC, NPAD, 128), jnp.float32),
        mesh=_sc_mesh(),
        scratch_types=[
            pltpu.VMEM_SHARED((NPAD, 128), jnp.float32),
            pltpu.VMEM((ch_r, B), jnp.int32),
            pltpu.VMEM((ch_r, B), jnp.int32),
            pltpu.VMEM((B, 128), jnp.float32),
            pltpu.VMEM((B, 128), jnp.float32),
            pltpu.VMEM((B, 128), jnp.float32),
            pltpu.SemaphoreType.DMA,
            pltpu.SemaphoreType.DMA,
            pltpu.SemaphoreType.DMA,
            pltpu.SemaphoreType.DMA,
            pltpu.SemaphoreType.DMA,
            pltpu.SemaphoreType.DMA,
        ],
    )


def _dinv_block(deg_ref):
    deg = deg_ref[0, :, 0:1] + deg_ref[1, :, 0:1] + 1.0
    return lax.rsqrt(deg)


def _tc0_body(x_ref, w_ref, deg_ref, out_ref):
    dinv = _dinv_block(deg_ref)
    h = jnp.dot(x_ref[...], w_ref[...], preferred_element_type=jnp.float32)
    hs = h * dinv
    out_ref[0] = hs[:, :128]
    out_ref[1] = hs[:, 128:]


def _tc1_body(a_ref, t_ref, deg_ref, b0_ref, w_ref, out_ref):
    dinv = _dinv_block(deg_ref)
    agg = jnp.concatenate([a_ref[0] + t_ref[0], a_ref[1] + t_ref[1]], axis=1)
    y = jnp.maximum(agg * dinv + b0_ref[...], 0.0)
    h = jnp.dot(y, w_ref[...], preferred_element_type=jnp.float32)
    out_ref[...] = h * dinv


def _tc2_body(a_ref, t_ref, deg_ref, b1_ref, out_ref):
    dinv = _dinv_block(deg_ref)
    z = (a_ref[0] + a_ref[1] + t_ref[...]) * dinv + b1_ref[...]
    m = jnp.max(z, axis=1, keepdims=True)
    ez = jnp.exp(z - m)
    lse = jnp.log(jnp.sum(ez, axis=1, keepdims=True))
    out_ref[...] = z - m - lse


def _row_spec(shape):
    if len(shape) == 3:
        return pl.BlockSpec((shape[0], MBLK, shape[2]), lambda i: (0, i, 0))
    return pl.BlockSpec((MBLK, shape[1]), lambda i: (i, 0))


def _full_spec(shape):
    return pl.BlockSpec(shape, lambda i: tuple(0 for _ in shape))


def _tc_call(body, in_arrays, full_mask, out_shape):
    in_specs = [
        _full_spec(a.shape) if f else _row_spec(a.shape)
        for a, f in zip(in_arrays, full_mask)
    ]
    return pl.pallas_call(
        body,
        grid=(N // MBLK,),
        in_specs=in_specs,
        out_specs=_row_spec(out_shape),
        out_shape=jax.ShapeDtypeStruct(out_shape, jnp.float32),
    )(*in_arrays)


def kernel(x, edge_index, W0, b0, W1, b1):
    src = edge_index[0].astype(jnp.int32)
    dst = edge_index[1].astype(jnp.int32)
    npad = EPAD - E_RAW
    pad_src = jnp.arange(npad, dtype=jnp.int32) % N
    pad_dst = N + (jnp.arange(npad, dtype=jnp.int32) % (NPAD - N))
    src = jnp.concatenate([src, pad_src])
    dst = jnp.concatenate([dst, pad_dst])
    dst_w = dst.reshape(NC * NS, ROWS_PER_WORKER, B)
    dst_s = dst.reshape(NS, KCH0, CHR0, B)
    src2 = jnp.stack([src, src + N]).reshape(NC, NS, KCH0, CHR0, B)
    src_e = src.reshape(NC * NS, KCH1, CHR1, B)
    dst_e = dst.reshape(NC * NS, KCH1, CHR1, B)

    ones128 = jnp.ones((B, 128), jnp.float32)
    zeros128 = jnp.zeros((ACC_PER_SUB, 128), jnp.float32)

    deg = _make_deg_kernel()(dst_w, ones128, zeros128)
    degn = deg[:, :N, :16]

    hs0 = _tc_call(_tc0_body, [x, W0, degn], [False, True, False],
                   (2, N, 128))
    table0 = hs0.reshape(2 * N, 128)
    agg0 = _make_agg_kernel(KCH0, CHR0, True)(table0, src2, dst_s, zeros128)

    table1 = _tc_call(_tc1_body,
                      [agg0[:, :N, :], hs0, degn, b0.reshape(1, 256), W1],
                      [False, False, False, True, True], (N, 128))
    agg1 = _make_agg_kernel(KCH1, CHR1, False)(table1, src_e, dst_e, zeros128)

    out = _tc_call(_tc2_body,
                   [agg1[:, :N, :], table1, degn, b1.reshape(1, 128)],
                   [False, False, False, True], (N, 128))
    return out


# speedup vs baseline: 2.5988x; 1.0442x over previous
import jax
import jax.numpy as jnp
from jax import lax
from jax.experimental import pallas as pl
from jax.experimental.pallas import tpu as pltpu
from jax.experimental.pallas import tpu_sc as plsc

N = 10000
NPAD = 10240
E_RAW = 320000
B = 112
EPAD = 322560
ROWS = EPAD // B
NC, NS = 2, 16
ROWS_PER_SUB = ROWS // NS
KCH0, CHR0 = 12, 15
ROWS_PER_WORKER = ROWS // (NC * NS)
KCH1, CHR1 = 9, 10
ACC_PER_SUB = NPAD // NS
MBLK = 1000


def _sc_mesh():
    return plsc.VectorSubcoreMesh(core_axis_name="c", subcore_axis_name="s",
                                  num_cores=NC, num_subcores=NS)


def _deg_body(dst_hbm, ones_hbm, zeros_hbm, deg_hbm, acc, dst_v, ones_v, sem):
    c = lax.axis_index("c")
    s = lax.axis_index("s")
    w = s * NC + c
    pltpu.sync_copy(zeros_hbm, acc.at[pl.ds(s * ACC_PER_SUB, ACC_PER_SUB)])
    pltpu.sync_copy(ones_hbm, ones_v)
    pltpu.sync_copy(dst_hbm.at[w], dst_v)
    plsc.subcore_barrier()

    def issue(j, _):
        pltpu.async_copy(ones_v, acc.at[dst_v.at[j]], sem, add=True)
        return ()

    def drain(j, _):
        pltpu.make_async_copy(ones_v, acc.at[dst_v.at[j]], sem).wait()
        return ()

    lax.fori_loop(0, ROWS_PER_WORKER, issue, ())
    lax.fori_loop(0, ROWS_PER_WORKER, drain, ())
    plsc.subcore_barrier()
    pltpu.sync_copy(acc.at[pl.ds(s * ACC_PER_SUB, ACC_PER_SUB)],
                    deg_hbm.at[c, pl.ds(s * ACC_PER_SUB, ACC_PER_SUB)])


def _make_deg_kernel():
    return pl.kernel(
        _deg_body,
        out_type=jax.ShapeDtypeStruct((NC, NPAD, 128), jnp.float32),
        mesh=_sc_mesh(),
        scratch_types=[
            pltpu.VMEM_SHARED((NPAD, 128), jnp.float32),
            pltpu.VMEM((ROWS_PER_WORKER, B), jnp.int32),
            pltpu.VMEM((B, 128), jnp.float32),
            pltpu.SemaphoreType.DMA,
        ],
    )


NBUF = 3


def _pipelined_chunk(table_hbm, acc, src_v, dst_v, bufs, gsems, ssems, ch_r):
    def g_issue(j):
        pltpu.async_copy(table_hbm.at[src_v.at[j]], bufs[j % NBUF],
                         gsems[j % NBUF])

    def g_wait(j):
        pltpu.make_async_copy(table_hbm.at[src_v.at[j]], bufs[j % NBUF],
                              gsems[j % NBUF]).wait()

    def s_issue(j):
        pltpu.async_copy(bufs[j % NBUF], acc.at[dst_v.at[j]], ssems[j % NBUF],
                         add=True)

    def s_wait(j):
        pltpu.make_async_copy(bufs[j % NBUF], acc.at[dst_v.at[j]],
                              ssems[j % NBUF]).wait()

    for j in range(NBUF):
        g_issue(j)
    for j in range(ch_r):
        g_wait(j)
        s_issue(j)
        if j + NBUF < ch_r:
            s_wait(j)
            g_issue(j + NBUF)
    for j in range(max(ch_r - NBUF, 0), ch_r):
        s_wait(j)


def _agg_body(kch, ch_r, col_split, table_hbm, src_hbm, dst_hbm, zeros_hbm,
              out_hbm, acc, src_v, dst_v, r0, r1, r2, gs0, gs1, gs2,
              ss0, ss1, ss2):
    c = lax.axis_index("c")
    s = lax.axis_index("s")
    w = s * NC + c
    pltpu.sync_copy(zeros_hbm, acc.at[pl.ds(s * ACC_PER_SUB, ACC_PER_SUB)])
    plsc.subcore_barrier()

    def chunk(k, _):
        if col_split:
            pltpu.sync_copy(src_hbm.at[c, s, k], src_v)
            pltpu.sync_copy(dst_hbm.at[s, k], dst_v)
        else:
            pltpu.sync_copy(src_hbm.at[w, k], src_v)
            pltpu.sync_copy(dst_hbm.at[w, k], dst_v)
        _pipelined_chunk(table_hbm, acc, src_v, dst_v, (r0, r1, r2),
                         (gs0, gs1, gs2), (ss0, ss1, ss2), ch_r)
        return ()

    lax.fori_loop(0, kch, chunk, ())
    plsc.subcore_barrier()
    pltpu.sync_copy(acc.at[pl.ds(s * ACC_PER_SUB, ACC_PER_SUB)],
                    out_hbm.at[c, pl.ds(s * ACC_PER_SUB, ACC_PER_SUB)])


def _make_agg_kernel(kch, ch_r, col_split):
    import functools
    return pl.kernel(
        functools.partial(_agg_body, kch, ch_r, col_split),
        out_type=jax.ShapeDtypeStruct((NC, NPAD, 128), jnp.float32),
        mesh=_sc_mesh(),
        scratch_types=[
            pltpu.VMEM_SHARED((NPAD, 128), jnp.float32),
            pltpu.VMEM((ch_r, B), jnp.int32),
            pltpu.VMEM((ch_r, B), jnp.int32),
            pltpu.VMEM((B, 128), jnp.float32),
            pltpu.VMEM((B, 128), jnp.float32),
            pltpu.VMEM((B, 128), jnp.float32),
            pltpu.SemaphoreType.DMA,
            pltpu.SemaphoreType.DMA,
            pltpu.SemaphoreType.DMA,
            pltpu.SemaphoreType.DMA,
            pltpu.SemaphoreType.DMA,
            pltpu.SemaphoreType.DMA,
        ],
    )


def _dinv_block(deg_ref):
    deg = deg_ref[0, :, 0:1] + deg_ref[1, :, 0:1] + 1.0
    return lax.rsqrt(deg)


def _tc0_body(x_ref, w_ref, deg_ref, out_ref):
    dinv = _dinv_block(deg_ref)
    h = jnp.dot(x_ref[...], w_ref[...], preferred_element_type=jnp.float32)
    hs = h * dinv
    out_ref[0] = hs[:, :128]
    out_ref[1] = hs[:, 128:]


def _tc1_body(a_ref, t_ref, deg_ref, b0_ref, w_ref, out_ref):
    dinv = _dinv_block(deg_ref)
    agg = jnp.concatenate([a_ref[0] + t_ref[0], a_ref[1] + t_ref[1]], axis=1)
    y = jnp.maximum(agg * dinv + b0_ref[...], 0.0)
    h = jnp.dot(y, w_ref[...], preferred_element_type=jnp.float32)
    out_ref[...] = h * dinv


def _tc2_body(a_ref, t_ref, deg_ref, b1_ref, out_ref):
    dinv = _dinv_block(deg_ref)
    z = (a_ref[0] + a_ref[1] + t_ref[...]) * dinv + b1_ref[...]
    m = jnp.max(z, axis=1, keepdims=True)
    ez = jnp.exp(z - m)
    lse = jnp.log(jnp.sum(ez, axis=1, keepdims=True))
    out_ref[...] = z - m - lse


def _row_spec(shape):
    if len(shape) == 3:
        return pl.BlockSpec((shape[0], MBLK, shape[2]), lambda i: (0, i, 0))
    return pl.BlockSpec((MBLK, shape[1]), lambda i: (i, 0))


def _full_spec(shape):
    return pl.BlockSpec(shape, lambda i: tuple(0 for _ in shape))


def _tc_call(body, in_arrays, full_mask, out_shape):
    in_specs = [
        _full_spec(a.shape) if f else _row_spec(a.shape)
        for a, f in zip(in_arrays, full_mask)
    ]
    return pl.pallas_call(
        body,
        grid=(N // MBLK,),
        in_specs=in_specs,
        out_specs=_row_spec(out_shape),
        out_shape=jax.ShapeDtypeStruct(out_shape, jnp.float32),
    )(*in_arrays)


def kernel(x, edge_index, W0, b0, W1, b1):
    src = edge_index[0].astype(jnp.int32)
    dst = edge_index[1].astype(jnp.int32)
    npad = EPAD - E_RAW
    pad_src = jnp.arange(npad, dtype=jnp.int32) % N
    pad_dst = N + (jnp.arange(npad, dtype=jnp.int32) % (NPAD - N))
    src = jnp.concatenate([src, pad_src])
    dst = jnp.concatenate([dst, pad_dst])
    dst_w = dst.reshape(NC * NS, ROWS_PER_WORKER, B)
    dst_s = dst.reshape(NS, KCH0, CHR0, B)
    src2 = jnp.stack([src, src + N]).reshape(NC, NS, KCH0, CHR0, B)
    src_e = src.reshape(NC * NS, KCH1, CHR1, B)
    dst_e = dst.reshape(NC * NS, KCH1, CHR1, B)

    ones128 = jnp.ones((B, 128), jnp.float32)
    zeros128 = jnp.zeros((ACC_PER_SUB, 128), jnp.float32)

    deg = _make_deg_kernel()(dst_w, ones128, zeros128)
    degn = deg[:, :, :16]

    hs0 = _tc_call(_tc0_body, [x, W0, degn], [False, True, False],
                   (2, N, 128))
    table0 = hs0.reshape(2 * N, 128)
    agg0 = _make_agg_kernel(KCH0, CHR0, True)(table0, src2, dst_s, zeros128)

    table1 = _tc_call(_tc1_body,
                      [agg0, hs0, degn, b0.reshape(1, 256), W1],
                      [False, False, False, True, True], (N, 128))
    agg1 = _make_agg_kernel(KCH1, CHR1, False)(table1, src_e, dst_e, zeros128)

    out = _tc_call(_tc2_body,
                   [agg1, table1, degn, b1.reshape(1, 128)],
                   [False, False, False, True], (N, 128))
    return out


# speedup vs baseline: 2.7033x; 1.0402x over previous
import jax
import jax.numpy as jnp
from jax import lax
from jax.experimental import pallas as pl
from jax.experimental.pallas import tpu as pltpu
from jax.experimental.pallas import tpu_sc as plsc

N = 10000
NPAD = 10112
E_RAW = 320000
B = 112
EPAD = 322560
ROWS = EPAD // B
NC, NS = 2, 16
ROWS_PER_SUB = ROWS // NS
KCH0, CHR0 = 9, 20
ROWS_PER_WORKER = ROWS // (NC * NS)
KCH1, CHR1 = 6, 15
ACC_PER_SUB = NPAD // NS
MBLK = 1000


def _sc_mesh():
    return plsc.VectorSubcoreMesh(core_axis_name="c", subcore_axis_name="s",
                                  num_cores=NC, num_subcores=NS)


def _deg_body(dst_hbm, ones_hbm, zeros_hbm, deg_hbm, acc, dst_v, ones_v, sem):
    c = lax.axis_index("c")
    s = lax.axis_index("s")
    w = s * NC + c
    pltpu.sync_copy(zeros_hbm, acc.at[pl.ds(s * ACC_PER_SUB, ACC_PER_SUB)])
    pltpu.sync_copy(ones_hbm, ones_v)
    pltpu.sync_copy(dst_hbm.at[w], dst_v)
    plsc.subcore_barrier()

    def issue(j, _):
        pltpu.async_copy(ones_v, acc.at[dst_v.at[j]], sem, add=True)
        return ()

    def drain(j, _):
        pltpu.make_async_copy(ones_v, acc.at[dst_v.at[j]], sem).wait()
        return ()

    lax.fori_loop(0, ROWS_PER_WORKER, issue, ())
    lax.fori_loop(0, ROWS_PER_WORKER, drain, ())
    plsc.subcore_barrier()
    pltpu.sync_copy(acc.at[pl.ds(s * ACC_PER_SUB, ACC_PER_SUB)],
                    deg_hbm.at[c, pl.ds(s * ACC_PER_SUB, ACC_PER_SUB)])


def _make_deg_kernel():
    return pl.kernel(
        _deg_body,
        out_type=jax.ShapeDtypeStruct((NC, NPAD, 128), jnp.float32),
        mesh=_sc_mesh(),
        scratch_types=[
            pltpu.VMEM_SHARED((NPAD, 128), jnp.float32),
            pltpu.VMEM((ROWS_PER_WORKER, B), jnp.int32),
            pltpu.VMEM((B, 128), jnp.float32),
            pltpu.SemaphoreType.DMA,
        ],
    )


NBUF = 3


def _pipelined_chunk(table_hbm, acc, src_v, dst_v, bufs, gsems, ssems, ch_r):
    def g_issue(j):
        pltpu.async_copy(table_hbm.at[src_v.at[j]], bufs[j % NBUF],
                         gsems[j % NBUF])

    def g_wait(j):
        pltpu.make_async_copy(table_hbm.at[src_v.at[j]], bufs[j % NBUF],
                              gsems[j % NBUF]).wait()

    def s_issue(j):
        pltpu.async_copy(bufs[j % NBUF], acc.at[dst_v.at[j]], ssems[j % NBUF],
                         add=True)

    def s_wait(j):
        pltpu.make_async_copy(bufs[j % NBUF], acc.at[dst_v.at[j]],
                              ssems[j % NBUF]).wait()

    for j in range(NBUF):
        g_issue(j)
    for j in range(ch_r):
        g_wait(j)
        s_issue(j)
        if j + NBUF < ch_r:
            s_wait(j)
            g_issue(j + NBUF)
    for j in range(max(ch_r - NBUF, 0), ch_r):
        s_wait(j)


def _agg_body(kch, ch_r, col_split, table_hbm, src_hbm, dst_hbm, zeros_hbm,
              out_hbm, acc, src_v, dst_v, r0, r1, r2, gs0, gs1, gs2,
              ss0, ss1, ss2):
    c = lax.axis_index("c")
    s = lax.axis_index("s")
    w = s * NC + c
    pltpu.sync_copy(zeros_hbm, acc.at[pl.ds(s * ACC_PER_SUB, ACC_PER_SUB)])
    plsc.subcore_barrier()

    def chunk(k, _):
        if col_split:
            pltpu.sync_copy(src_hbm.at[c, s, k], src_v)
            pltpu.sync_copy(dst_hbm.at[s, k], dst_v)
        else:
            pltpu.sync_copy(src_hbm.at[w, k], src_v)
            pltpu.sync_copy(dst_hbm.at[w, k], dst_v)
        _pipelined_chunk(table_hbm, acc, src_v, dst_v, (r0, r1, r2),
                         (gs0, gs1, gs2), (ss0, ss1, ss2), ch_r)
        return ()

    lax.fori_loop(0, kch, chunk, ())
    plsc.subcore_barrier()
    pltpu.sync_copy(acc.at[pl.ds(s * ACC_PER_SUB, ACC_PER_SUB)],
                    out_hbm.at[c, pl.ds(s * ACC_PER_SUB, ACC_PER_SUB)])


def _make_agg_kernel(kch, ch_r, col_split):
    import functools
    return pl.kernel(
        functools.partial(_agg_body, kch, ch_r, col_split),
        out_type=jax.ShapeDtypeStruct((NC, NPAD, 128), jnp.float32),
        mesh=_sc_mesh(),
        scratch_types=[
            pltpu.VMEM_SHARED((NPAD, 128), jnp.float32),
            pltpu.VMEM((ch_r, B), jnp.int32),
            pltpu.VMEM((ch_r, B), jnp.int32),
            pltpu.VMEM((B, 128), jnp.float32),
            pltpu.VMEM((B, 128), jnp.float32),
            pltpu.VMEM((B, 128), jnp.float32),
            pltpu.SemaphoreType.DMA,
            pltpu.SemaphoreType.DMA,
            pltpu.SemaphoreType.DMA,
            pltpu.SemaphoreType.DMA,
            pltpu.SemaphoreType.DMA,
            pltpu.SemaphoreType.DMA,
        ],
    )


def _dinv_block(deg_ref):
    deg = deg_ref[0, :, 0:1] + deg_ref[1, :, 0:1] + 1.0
    return lax.rsqrt(deg)


def _tc0_body(x_ref, w_ref, deg_ref, out_ref):
    dinv = _dinv_block(deg_ref)
    h = jnp.dot(x_ref[...], w_ref[...], preferred_element_type=jnp.float32)
    hs = h * dinv
    out_ref[0] = hs[:, :128]
    out_ref[1] = hs[:, 128:]


def _tc1_body(a_ref, t_ref, deg_ref, b0_ref, w_ref, out_ref):
    dinv = _dinv_block(deg_ref)
    agg = jnp.concatenate([a_ref[0] + t_ref[0], a_ref[1] + t_ref[1]], axis=1)
    y = jnp.maximum(agg * dinv + b0_ref[...], 0.0)
    h = jnp.dot(y, w_ref[...], preferred_element_type=jnp.float32)
    out_ref[...] = h * dinv


def _tc2_body(a_ref, t_ref, deg_ref, b1_ref, out_ref):
    dinv = _dinv_block(deg_ref)
    z = (a_ref[0] + a_ref[1] + t_ref[...]) * dinv + b1_ref[...]
    m = jnp.max(z, axis=1, keepdims=True)
    ez = jnp.exp(z - m)
    lse = jnp.log(jnp.sum(ez, axis=1, keepdims=True))
    out_ref[...] = z - m - lse


def _row_spec(shape):
    if len(shape) == 3:
        return pl.BlockSpec((shape[0], MBLK, shape[2]), lambda i: (0, i, 0))
    return pl.BlockSpec((MBLK, shape[1]), lambda i: (i, 0))


def _full_spec(shape):
    return pl.BlockSpec(shape, lambda i: tuple(0 for _ in shape))


def _tc_call(body, in_arrays, full_mask, out_shape):
    in_specs = [
        _full_spec(a.shape) if f else _row_spec(a.shape)
        for a, f in zip(in_arrays, full_mask)
    ]
    return pl.pallas_call(
        body,
        grid=(N // MBLK,),
        in_specs=in_specs,
        out_specs=_row_spec(out_shape),
        out_shape=jax.ShapeDtypeStruct(out_shape, jnp.float32),
    )(*in_arrays)


def kernel(x, edge_index, W0, b0, W1, b1):
    src = edge_index[0].astype(jnp.int32)
    dst = edge_index[1].astype(jnp.int32)
    npad = EPAD - E_RAW
    pad_src = jnp.arange(npad, dtype=jnp.int32) % N
    pad_dst = N + (jnp.arange(npad, dtype=jnp.int32) % (NPAD - N))
    src = jnp.concatenate([src, pad_src])
    dst = jnp.concatenate([dst, pad_dst])
    dst_w = dst.reshape(NC * NS, ROWS_PER_WORKER, B)
    dst_s = dst.reshape(NS, KCH0, CHR0, B)
    src2 = jnp.stack([src, src + N]).reshape(NC, NS, KCH0, CHR0, B)
    src_e = src.reshape(NC * NS, KCH1, CHR1, B)
    dst_e = dst.reshape(NC * NS, KCH1, CHR1, B)

    ones128 = jnp.ones((B, 128), jnp.float32)
    zeros128 = jnp.zeros((ACC_PER_SUB, 128), jnp.float32)

    deg = _make_deg_kernel()(dst_w, ones128, zeros128)
    degn = deg[:, :, :16]

    hs0 = _tc_call(_tc0_body, [x, W0, degn], [False, True, False],
                   (2, N, 128))
    table0 = hs0.reshape(2 * N, 128)
    agg0 = _make_agg_kernel(KCH0, CHR0, True)(table0, src2, dst_s, zeros128)

    table1 = _tc_call(_tc1_body,
                      [agg0, hs0, degn, b0.reshape(1, 256), W1],
                      [False, False, False, True, True], (N, 128))
    agg1 = _make_agg_kernel(KCH1, CHR1, False)(table1, src_e, dst_e, zeros128)

    out = _tc_call(_tc2_body,
                   [agg1, table1, degn, b1.reshape(1, 128)],
                   [False, False, False, True], (N, 128))
    return out
